# baseline (speedup 1.0000x reference)
import functools

import jax
import jax.numpy as jnp
from jax import lax
from jax.experimental import pallas as pl
from jax.experimental.pallas import tpu as pltpu
from jax.experimental.pallas import tpu_sc as plsc

B = 16384
D = 32
NROWS = 1000000

_info = plsc.get_sparse_core_info()
NC = _info.num_cores
NS = _info.num_subcores
L = _info.num_lanes
NW = NC * NS
BPW = B // NW
NG = BPW // L

_mesh = plsc.VectorSubcoreMesh(core_axis_name="c", subcore_axis_name="s")


@functools.partial(
    pl.kernel,
    mesh=_mesh,
    compiler_params=pltpu.CompilerParams(
        needs_layout_passes=False, use_tc_tiling_on_sc=False),
    out_type=jax.ShapeDtypeStruct((B,), jnp.float32),
    scratch_types=[
        pltpu.VMEM((BPW,), jnp.int32),
        pltpu.VMEM((BPW,), jnp.int32),
        pltpu.VMEM((D, BPW), jnp.float32),
        pltpu.VMEM((D, BPW), jnp.float32),
        pltpu.VMEM((BPW,), jnp.float32),
        pltpu.VMEM((BPW,), jnp.float32),
        pltpu.VMEM((BPW,), jnp.float32),
        pltpu.VMEM((BPW,), jnp.float32),
        pltpu.SemaphoreType.DMA,
        pltpu.SemaphoreType.DMA,
    ],
)
def _mf_kernel(uid_hbm, iid_hbm, avg_hbm, p_hbm, q_hbm, ub_hbm, ib_hbm,
               out_hbm, uidx, iidx, pdv, qdv, ubv, ibv, avgv, outv,
               gsem, bsem):
    wid = lax.axis_index("s") * NC + lax.axis_index("c")
    base = wid * BPW

    pltpu.sync_copy(uid_hbm.at[pl.ds(base, BPW)], uidx)
    pltpu.sync_copy(iid_hbm.at[pl.ds(base, BPW)], iidx)
    cu = pltpu.async_copy(ub_hbm.at[0].at[uidx], ubv, bsem)
    ci = pltpu.async_copy(ib_hbm.at[0].at[iidx], ibv, bsem)

    copies = []
    for d in range(D):
        copies.append(
            pltpu.async_copy(p_hbm.at[d].at[uidx], pdv.at[d], gsem))
        copies.append(
            pltpu.async_copy(q_hbm.at[d].at[iidx], qdv.at[d], gsem))

    pltpu.sync_copy(avg_hbm.at[0, pl.ds(base, BPW)], avgv)
    cu.wait()
    ci.wait()
    for c in copies:
        c.wait()

    def body(g, _):
        o = g * L
        acc = avgv[pl.ds(o, L)] + ubv[pl.ds(o, L)] + ibv[pl.ds(o, L)]
        for d in range(D):
            acc += pdv[d, pl.ds(o, L)] * qdv[d, pl.ds(o, L)]
        outv[pl.ds(o, L)] = acc
        return 0

    lax.fori_loop(0, NG, body, 0)
    pltpu.sync_copy(outv, out_hbm.at[pl.ds(base, BPW)])


def kernel(user_id, item_id, avg_score, p, q, user_bias, item_bias):
    out = _mf_kernel(user_id, item_id, avg_score.T, p.T, q.T,
                     user_bias.T, item_bias.T)
    return out.reshape(B, 1)


# speedup vs baseline: 11.1590x; 11.1590x over previous
import functools

import jax
import jax.numpy as jnp
from jax import lax
from jax.experimental import pallas as pl
from jax.experimental.pallas import tpu as pltpu
from jax.experimental.pallas import tpu_sc as plsc

B = 16384
D = 32
NROWS = 1000000

_info = plsc.get_sparse_core_info()
NC = _info.num_cores
NS = _info.num_subcores
L = _info.num_lanes
NW = NC * NS
BPW = B // NW

RNG = 31232
WL = 31872
X = 18432
CAP = 704
BUF = 832
NCH = B // L
DH = D // 2

_mesh = plsc.VectorSubcoreMesh(core_axis_name="c", subcore_axis_name="s")


@functools.partial(
    pl.kernel,
    mesh=_mesh,
    compiler_params=pltpu.CompilerParams(needs_layout_passes=False),
    out_type=(
        jax.ShapeDtypeStruct((D * X,), jnp.float32),
        jax.ShapeDtypeStruct((D * X,), jnp.float32),
        jax.ShapeDtypeStruct((D * X,), jnp.float32),
        jax.ShapeDtypeStruct((D * X,), jnp.float32),
        jax.ShapeDtypeStruct((X,), jnp.float32),
        jax.ShapeDtypeStruct((X,), jnp.float32),
        jax.ShapeDtypeStruct((X,), jnp.float32),
        jax.ShapeDtypeStruct((X,), jnp.float32),
    ),
    scratch_types=[
        pltpu.VMEM((WL,), jnp.float32),
        pltpu.VMEM((WL,), jnp.float32),
        pltpu.VMEM((B,), jnp.int32),
        pltpu.VMEM((B,), jnp.int32),
        pltpu.VMEM((BUF,), jnp.int32),
        pltpu.VMEM((BUF,), jnp.int32),
        pltpu.VMEM((BUF,), jnp.int32),
        pltpu.VMEM((BUF,), jnp.int32),
        pltpu.VMEM((CAP,), jnp.int32),
        pltpu.VMEM((CAP,), jnp.float32),
        pltpu.VMEM((CAP,), jnp.int32),
        pltpu.VMEM((CAP,), jnp.float32),
        pltpu.VMEM((2304,), jnp.float32),
        pltpu.VMEM_SHARED((DH * X,), jnp.float32),
        pltpu.VMEM_SHARED((X,), jnp.float32),
        pltpu.SemaphoreType.DMA,
        pltpu.SemaphoreType.DMA,
        pltpu.SemaphoreType.DMA,
        pltpu.SemaphoreType.DMA,
    ],
)
def _scan_kernel(uid_hbm, iid_hbm, p_hbm, q_hbm, ub_hbm, ib_hbm,
                 pv0, pv1, qv0, qv1, ubp0, ubp1, ibp0, ibp1,
                 win0, win1, uidl, iidl, offs_u, bs_u, offs_i, bs_i,
                 posa, valsa, posb, valsb, zbuf, part, partb,
                 sem0, sem1, sca, scb):
    cid = lax.axis_index("c")
    sid = lax.axis_index("s")
    wid = sid * NC + cid
    lo = pl.multiple_of(wid * RNG, 128)
    hi = jnp.where(wid == NW - 1, NROWS, lo + RNG)
    lane = lax.iota(jnp.int32, L)
    shp = (DH * X) // NS
    bshp = X // NS

    def zero(g, _):
        zbuf[pl.ds(g * L, L)] = jnp.zeros((L,), jnp.float32)
        return 0
    lax.fori_loop(0, 2304 // L, zero, 0)

    pltpu.sync_copy(uid_hbm, uidl)
    pltpu.sync_copy(iid_hbm, iidl)

    def prefill(j, _):
        o = j * L
        dump = B + lax.rem(wid * 61 + o + lane, 2048)
        bs_u[pl.ds(o, L)] = dump
        bs_i[pl.ds(o, L)] = dump
        offs_u[pl.ds(o, L)] = jnp.zeros((L,), jnp.int32)
        offs_i[pl.ds(o, L)] = jnp.zeros((L,), jnp.int32)
        return 0
    lax.fori_loop(0, BUF // L, prefill, 0)

    def scan(g, carry):
        cu, ci = carry
        o = g * L
        u = uidl[pl.ds(o, L)]
        i = iidl[pl.ds(o, L)]
        b = o + lane
        mu = (u >= lo) & (u < hi)
        mi = (i >= lo) & (i < hi)
        plsc.store_compressed(offs_u.at[pl.ds(cu, L)], u - lo, mask=mu)
        plsc.store_compressed(bs_u.at[pl.ds(cu, L)], b, mask=mu)
        plsc.store_compressed(offs_i.at[pl.ds(ci, L)], i - lo, mask=mi)
        plsc.store_compressed(bs_i.at[pl.ds(ci, L)], b, mask=mi)
        cu = cu + plsc.all_reduce_population_count(mu)[0]
        ci = ci + plsc.all_reduce_population_count(mi)[0]
        return cu, ci
    lax.fori_loop(0, NCH, scan, (jnp.int32(0), jnp.int32(0)))

    def extract(win, offs, bs, pos, vals, dbase):
        for j in range(CAP // L):
            o = j * L
            idx = offs[pl.ds(o, L)]
            vals[pl.ds(o, L)] = plsc.load_gather(win, [idx])
            pos[pl.ds(o, L)] = bs[pl.ds(o, L)] + dbase

    def phase(tbl, bias, offs, bs, tout0, tout1, bout0, bout1, d0, do_bias):
        for c in range(shp // 2304):
            pltpu.sync_copy(zbuf, part.at[pl.ds(sid * shp + c * 2304, 2304)])
        if do_bias:
            pltpu.sync_copy(zbuf.at[pl.ds(0, bshp)],
                            partb.at[pl.ds(sid * bshp, bshp)])
        plsc.subcore_barrier()

        c0 = pltpu.async_copy(tbl.at[d0, 0, pl.ds(lo, WL)], win0, sem0)
        c1 = pltpu.async_copy(tbl.at[d0 + 1, 0, pl.ds(lo, WL)], win1, sem1)
        c0.wait()
        extract(win0, offs, bs, posa, valsa, 0)
        pltpu.async_copy(valsa, part.at[posa], sca)
        pltpu.async_copy(tbl.at[d0 + 2, 0, pl.ds(lo, WL)], win0, sem0)
        c1.wait()
        extract(win1, offs, bs, posb, valsb, X)
        pltpu.async_copy(valsb, part.at[posb], scb)
        pltpu.async_copy(tbl.at[d0 + 3, 0, pl.ds(lo, WL)], win1, sem1)

        def slab(dd, _):
            @pl.when(lax.rem(dd, 2) == 0)
            def _():
                pltpu.make_async_copy(valsa, part.at[posa], sca).wait()
                pltpu.make_async_copy(
                    tbl.at[0, 0, pl.ds(lo, WL)], win0, sem0).wait()
                extract(win0, offs, bs, posa, valsa, dd * X)
                pltpu.async_copy(valsa, part.at[posa], sca)

                @pl.when(dd < DH - 2)
                def _():
                    pltpu.async_copy(
                        tbl.at[d0 + dd + 2, 0, pl.ds(lo, WL)], win0, sem0)

            @pl.when(lax.rem(dd, 2) == 1)
            def _():
                pltpu.make_async_copy(valsb, part.at[posb], scb).wait()
                pltpu.make_async_copy(
                    tbl.at[0, 0, pl.ds(lo, WL)], win1, sem1).wait()
                extract(win1, offs, bs, posb, valsb, dd * X)
                pltpu.async_copy(valsb, part.at[posb], scb)

                @pl.when(dd < DH - 2)
                def _():
                    pltpu.async_copy(
                        tbl.at[d0 + dd + 2, 0, pl.ds(lo, WL)], win1, sem1)
            return 0
        lax.fori_loop(2, DH, slab, 0)

        pltpu.make_async_copy(valsa, part.at[posa], sca).wait()
        if do_bias:
            cbias = pltpu.async_copy(bias.at[0, pl.ds(lo, WL)], win0, sem0)
            cbias.wait()
            extract(win0, offs, bs, posa, valsa, 0)
            ca = pltpu.async_copy(valsa, partb.at[posa], sca)
        pltpu.make_async_copy(valsb, part.at[posb], scb).wait()
        if do_bias:
            ca.wait()
        plsc.subcore_barrier()

        @pl.when(cid == 0)
        def _():
            pltpu.sync_copy(part.at[pl.ds(sid * shp, shp)],
                            tout0.at[pl.ds(d0 * X + sid * shp, shp)])
            if do_bias:
                pltpu.sync_copy(partb.at[pl.ds(sid * bshp, bshp)],
                                bout0.at[pl.ds(sid * bshp, bshp)])

        @pl.when(cid == 1)
        def _():
            pltpu.sync_copy(part.at[pl.ds(sid * shp, shp)],
                            tout1.at[pl.ds(d0 * X + sid * shp, shp)])
            if do_bias:
                pltpu.sync_copy(partb.at[pl.ds(sid * bshp, bshp)],
                                bout1.at[pl.ds(sid * bshp, bshp)])
        plsc.subcore_barrier()

    phase(p_hbm, ub_hbm, offs_u, bs_u, pv0, pv1, ubp0, ubp1, 0, True)
    phase(p_hbm, ub_hbm, offs_u, bs_u, pv0, pv1, ubp0, ubp1, DH, False)
    phase(q_hbm, ib_hbm, offs_i, bs_i, qv0, qv1, ibp0, ibp1, 0, True)
    phase(q_hbm, ib_hbm, offs_i, bs_i, qv0, qv1, ibp0, ibp1, DH, False)


@functools.partial(
    pl.kernel,
    mesh=_mesh,
    compiler_params=pltpu.CompilerParams(needs_layout_passes=False),
    out_type=jax.ShapeDtypeStruct((B,), jnp.float32),
    scratch_types=[
        pltpu.VMEM((D, BPW), jnp.float32),
        pltpu.VMEM((D, BPW), jnp.float32),
        pltpu.VMEM((D, BPW), jnp.float32),
        pltpu.VMEM((D, BPW), jnp.float32),
        pltpu.VMEM((BPW,), jnp.float32),
        pltpu.VMEM((BPW,), jnp.float32),
        pltpu.VMEM((BPW,), jnp.float32),
        pltpu.VMEM((BPW,), jnp.float32),
        pltpu.VMEM((BPW,), jnp.float32),
        pltpu.VMEM((BPW,), jnp.float32),
        pltpu.SemaphoreType.DMA,
    ],
)
def _merge_kernel(avg_hbm, pv0, pv1, qv0, qv1, ubp0, ubp1, ibp0, ibp1,
                  out_hbm, p0s, p1s, q0s, q1s, u0s, u1s, i0s, i1s,
                  avgv, outv, sem):
    wid = lax.axis_index("s") * NC + lax.axis_index("c")
    base = wid * BPW

    copies = []
    for d in range(D):
        copies.append(pltpu.async_copy(
            pv0.at[pl.ds(d * X + base, BPW)], p0s.at[d], sem))
        copies.append(pltpu.async_copy(
            pv1.at[pl.ds(d * X + base, BPW)], p1s.at[d], sem))
        copies.append(pltpu.async_copy(
            qv0.at[pl.ds(d * X + base, BPW)], q0s.at[d], sem))
        copies.append(pltpu.async_copy(
            qv1.at[pl.ds(d * X + base, BPW)], q1s.at[d], sem))
    copies.append(pltpu.async_copy(ubp0.at[pl.ds(base, BPW)], u0s, sem))
    copies.append(pltpu.async_copy(ubp1.at[pl.ds(base, BPW)], u1s, sem))
    copies.append(pltpu.async_copy(ibp0.at[pl.ds(base, BPW)], i0s, sem))
    copies.append(pltpu.async_copy(ibp1.at[pl.ds(base, BPW)], i1s, sem))
    pltpu.sync_copy(avg_hbm.at[0, pl.ds(base, BPW)], avgv)
    for c in copies:
        c.wait()

    def body(g, _):
        o = g * L
        acc = (avgv[pl.ds(o, L)]
               + u0s[pl.ds(o, L)] + u1s[pl.ds(o, L)]
               + i0s[pl.ds(o, L)] + i1s[pl.ds(o, L)])
        for d in range(D):
            acc += ((p0s[d, pl.ds(o, L)] + p1s[d, pl.ds(o, L)])
                    * (q0s[d, pl.ds(o, L)] + q1s[d, pl.ds(o, L)]))
        outv[pl.ds(o, L)] = acc
        return 0

    lax.fori_loop(0, BPW // L, body, 0)
    pltpu.sync_copy(outv, out_hbm.at[pl.ds(base, BPW)])


def kernel(user_id, item_id, avg_score, p, q, user_bias, item_bias):
    pv0, pv1, qv0, qv1, ub0, ub1, ib0, ib1 = _scan_kernel(
        user_id, item_id,
        p.T.reshape(D, 1, NROWS), q.T.reshape(D, 1, NROWS),
        user_bias.T, item_bias.T)
    out = _merge_kernel(avg_score.T, pv0, pv1, qv0, qv1,
                        ub0, ub1, ib0, ib1)
    return out.reshape(B, 1)


# speedup vs baseline: 28.0666x; 2.5152x over previous
import functools

import jax
import jax.numpy as jnp
from jax import lax
from jax.experimental import pallas as pl
from jax.experimental.pallas import tpu as pltpu
from jax.experimental.pallas import tpu_sc as plsc

B = 16384
D = 32
NROWS = 1000000

_info = plsc.get_sparse_core_info()
NC = _info.num_cores
NS = _info.num_subcores
L = _info.num_lanes
NW = NC * NS
BPW = B // NW

RNG = 31232
WL = 31872
X = 18432
CAP = 704
BUF = 832
NCH = B // L
DH = D // 2

_mesh = plsc.VectorSubcoreMesh(core_axis_name="c", subcore_axis_name="s")


@functools.partial(
    pl.kernel,
    mesh=_mesh,
    compiler_params=pltpu.CompilerParams(needs_layout_passes=False),
    out_type=(
        jax.ShapeDtypeStruct((D * X,), jnp.float32),
        jax.ShapeDtypeStruct((D * X,), jnp.float32),
        jax.ShapeDtypeStruct((D * X,), jnp.float32),
        jax.ShapeDtypeStruct((D * X,), jnp.float32),
        jax.ShapeDtypeStruct((X,), jnp.float32),
        jax.ShapeDtypeStruct((X,), jnp.float32),
        jax.ShapeDtypeStruct((X,), jnp.float32),
        jax.ShapeDtypeStruct((X,), jnp.float32),
    ),
    scratch_types=[
        pltpu.VMEM((WL,), jnp.float32),
        pltpu.VMEM((WL,), jnp.float32),
        pltpu.VMEM((B,), jnp.int32),
        pltpu.VMEM((B,), jnp.int32),
        pltpu.VMEM((BUF,), jnp.int32),
        pltpu.VMEM((BUF,), jnp.int32),
        pltpu.VMEM((BUF,), jnp.int32),
        pltpu.VMEM((BUF,), jnp.int32),
        pltpu.VMEM((CAP,), jnp.int32),
        pltpu.VMEM((CAP,), jnp.float32),
        pltpu.VMEM((CAP,), jnp.int32),
        pltpu.VMEM((CAP,), jnp.float32),
        pltpu.VMEM((2304,), jnp.float32),
        pltpu.VMEM_SHARED((DH * X,), jnp.float32),
        pltpu.VMEM_SHARED((X,), jnp.float32),
        pltpu.SemaphoreType.DMA,
        pltpu.SemaphoreType.DMA,
        pltpu.SemaphoreType.DMA,
        pltpu.SemaphoreType.DMA,
    ],
)
def _scan_kernel(uid_hbm, iid_hbm, p_hbm, q_hbm, ub_hbm, ib_hbm,
                 pv0, pv1, qv0, qv1, ubp0, ubp1, ibp0, ibp1,
                 win0, win1, uidl, iidl, offs_u, bs_u, offs_i, bs_i,
                 posa, valsa, posb, valsb, zbuf, part, partb,
                 sem0, sem1, sca, scb):
    cid = lax.axis_index("c")
    sid = lax.axis_index("s")
    wid = sid * NC + cid
    lo = pl.multiple_of(wid * RNG, 128)
    hi = jnp.where(wid == NW - 1, NROWS, lo + RNG)
    lane = lax.iota(jnp.int32, L)
    shp = (DH * X) // NS
    bshp = X // NS

    def zero(g, _):
        zbuf[pl.ds(g * L, L)] = jnp.zeros((L,), jnp.float32)
        return 0
    lax.fori_loop(0, 2304 // L, zero, 0)

    pltpu.sync_copy(uid_hbm, uidl)
    pltpu.sync_copy(iid_hbm, iidl)

    def prefill(j, _):
        o = j * L
        dump = B + lax.rem(wid * 61 + o + lane, 2048)
        bs_u[pl.ds(o, L)] = dump
        bs_i[pl.ds(o, L)] = dump
        offs_u[pl.ds(o, L)] = jnp.zeros((L,), jnp.int32)
        offs_i[pl.ds(o, L)] = jnp.zeros((L,), jnp.int32)
        return 0
    lax.fori_loop(0, BUF // L, prefill, 0)

    def scan(g, carry):
        cu, ci = carry
        o = g * L
        u = uidl[pl.ds(o, L)]
        i = iidl[pl.ds(o, L)]
        b = o + lane
        mu = (u >= lo) & (u < hi)
        mi = (i >= lo) & (i < hi)
        plsc.store_compressed(offs_u.at[pl.ds(cu, L)], u - lo, mask=mu)
        plsc.store_compressed(bs_u.at[pl.ds(cu, L)], b, mask=mu)
        plsc.store_compressed(offs_i.at[pl.ds(ci, L)], i - lo, mask=mi)
        plsc.store_compressed(bs_i.at[pl.ds(ci, L)], b, mask=mi)
        cu = cu + plsc.all_reduce_population_count(mu)[0]
        ci = ci + plsc.all_reduce_population_count(mi)[0]
        return cu, ci
    lax.fori_loop(0, NCH, scan, (jnp.int32(0), jnp.int32(0)))

    def extract(win, offs, bs, pos, vals, dbase):
        for j in range(CAP // L):
            o = j * L
            idx = offs[pl.ds(o, L)]
            vals[pl.ds(o, L)] = plsc.load_gather(win, [idx])
            pos[pl.ds(o, L)] = bs[pl.ds(o, L)] + dbase

    def phase(tbl, bias, offs, bs, tout0, tout1, bout0, bout1, d0, do_bias):
        for c in range(shp // 2304):
            pltpu.sync_copy(zbuf, part.at[pl.ds(sid * shp + c * 2304, 2304)])
        if do_bias:
            pltpu.sync_copy(zbuf.at[pl.ds(0, bshp)],
                            partb.at[pl.ds(sid * bshp, bshp)])
        plsc.subcore_barrier()

        c0 = pltpu.async_copy(tbl.at[d0, pl.ds(lo, WL)], win0, sem0)
        c1 = pltpu.async_copy(tbl.at[d0 + 1, pl.ds(lo, WL)], win1, sem1)
        c0.wait()
        extract(win0, offs, bs, posa, valsa, 0)
        pltpu.async_copy(valsa, part.at[posa], sca)
        pltpu.async_copy(tbl.at[d0 + 2, pl.ds(lo, WL)], win0, sem0)
        c1.wait()
        extract(win1, offs, bs, posb, valsb, X)
        pltpu.async_copy(valsb, part.at[posb], scb)
        pltpu.async_copy(tbl.at[d0 + 3, pl.ds(lo, WL)], win1, sem1)

        def slab(dd, _):
            @pl.when(lax.rem(dd, 2) == 0)
            def _():
                pltpu.make_async_copy(valsa, part.at[posa], sca).wait()
                pltpu.make_async_copy(
                    tbl.at[0, pl.ds(lo, WL)], win0, sem0).wait()
                extract(win0, offs, bs, posa, valsa, dd * X)
                pltpu.async_copy(valsa, part.at[posa], sca)

                @pl.when(dd < DH - 2)
                def _():
                    pltpu.async_copy(
                        tbl.at[d0 + dd + 2, pl.ds(lo, WL)], win0, sem0)

            @pl.when(lax.rem(dd, 2) == 1)
            def _():
                pltpu.make_async_copy(valsb, part.at[posb], scb).wait()
                pltpu.make_async_copy(
                    tbl.at[0, pl.ds(lo, WL)], win1, sem1).wait()
                extract(win1, offs, bs, posb, valsb, dd * X)
                pltpu.async_copy(valsb, part.at[posb], scb)

                @pl.when(dd < DH - 2)
                def _():
                    pltpu.async_copy(
                        tbl.at[d0 + dd + 2, pl.ds(lo, WL)], win1, sem1)
            return 0
        lax.fori_loop(2, DH, slab, 0)

        pltpu.make_async_copy(valsa, part.at[posa], sca).wait()
        if do_bias:
            cbias = pltpu.async_copy(bias.at[0, pl.ds(lo, WL)], win0, sem0)
            cbias.wait()
            extract(win0, offs, bs, posa, valsa, 0)
            ca = pltpu.async_copy(valsa, partb.at[posa], sca)
        pltpu.make_async_copy(valsb, part.at[posb], scb).wait()
        if do_bias:
            ca.wait()
        plsc.subcore_barrier()

        @pl.when(cid == 0)
        def _():
            pltpu.sync_copy(part.at[pl.ds(sid * shp, shp)],
                            tout0.at[pl.ds(d0 * X + sid * shp, shp)])
            if do_bias:
                pltpu.sync_copy(partb.at[pl.ds(sid * bshp, bshp)],
                                bout0.at[pl.ds(sid * bshp, bshp)])

        @pl.when(cid == 1)
        def _():
            pltpu.sync_copy(part.at[pl.ds(sid * shp, shp)],
                            tout1.at[pl.ds(d0 * X + sid * shp, shp)])
            if do_bias:
                pltpu.sync_copy(partb.at[pl.ds(sid * bshp, bshp)],
                                bout1.at[pl.ds(sid * bshp, bshp)])
        plsc.subcore_barrier()

    phase(p_hbm, ub_hbm, offs_u, bs_u, pv0, pv1, ubp0, ubp1, 0, True)
    phase(p_hbm, ub_hbm, offs_u, bs_u, pv0, pv1, ubp0, ubp1, DH, False)
    phase(q_hbm, ib_hbm, offs_i, bs_i, qv0, qv1, ibp0, ibp1, 0, True)
    phase(q_hbm, ib_hbm, offs_i, bs_i, qv0, qv1, ibp0, ibp1, DH, False)


@functools.partial(
    pl.kernel,
    mesh=_mesh,
    compiler_params=pltpu.CompilerParams(needs_layout_passes=False),
    out_type=jax.ShapeDtypeStruct((B,), jnp.float32),
    scratch_types=[
        pltpu.VMEM((D, BPW), jnp.float32),
        pltpu.VMEM((D, BPW), jnp.float32),
        pltpu.VMEM((D, BPW), jnp.float32),
        pltpu.VMEM((D, BPW), jnp.float32),
        pltpu.VMEM((BPW,), jnp.float32),
        pltpu.VMEM((BPW,), jnp.float32),
        pltpu.VMEM((BPW,), jnp.float32),
        pltpu.VMEM((BPW,), jnp.float32),
        pltpu.VMEM((BPW,), jnp.float32),
        pltpu.VMEM((BPW,), jnp.float32),
        pltpu.SemaphoreType.DMA,
    ],
)
def _merge_kernel(avg_hbm, pv0, pv1, qv0, qv1, ubp0, ubp1, ibp0, ibp1,
                  out_hbm, p0s, p1s, q0s, q1s, u0s, u1s, i0s, i1s,
                  avgv, outv, sem):
    wid = lax.axis_index("s") * NC + lax.axis_index("c")
    base = wid * BPW

    copies = []
    for d in range(D):
        copies.append(pltpu.async_copy(
            pv0.at[pl.ds(d * X + base, BPW)], p0s.at[d], sem))
        copies.append(pltpu.async_copy(
            pv1.at[pl.ds(d * X + base, BPW)], p1s.at[d], sem))
        copies.append(pltpu.async_copy(
            qv0.at[pl.ds(d * X + base, BPW)], q0s.at[d], sem))
        copies.append(pltpu.async_copy(
            qv1.at[pl.ds(d * X + base, BPW)], q1s.at[d], sem))
    copies.append(pltpu.async_copy(ubp0.at[pl.ds(base, BPW)], u0s, sem))
    copies.append(pltpu.async_copy(ubp1.at[pl.ds(base, BPW)], u1s, sem))
    copies.append(pltpu.async_copy(ibp0.at[pl.ds(base, BPW)], i0s, sem))
    copies.append(pltpu.async_copy(ibp1.at[pl.ds(base, BPW)], i1s, sem))
    pltpu.sync_copy(avg_hbm.at[0, pl.ds(base, BPW)], avgv)
    for c in copies:
        c.wait()

    def body(g, _):
        o = g * L
        acc = (avgv[pl.ds(o, L)]
               + u0s[pl.ds(o, L)] + u1s[pl.ds(o, L)]
               + i0s[pl.ds(o, L)] + i1s[pl.ds(o, L)])
        for d in range(D):
            acc += ((p0s[d, pl.ds(o, L)] + p1s[d, pl.ds(o, L)])
                    * (q0s[d, pl.ds(o, L)] + q1s[d, pl.ds(o, L)]))
        outv[pl.ds(o, L)] = acc
        return 0

    lax.fori_loop(0, BPW // L, body, 0)
    pltpu.sync_copy(outv, out_hbm.at[pl.ds(base, BPW)])


def kernel(user_id, item_id, avg_score, p, q, user_bias, item_bias):
    pv0, pv1, qv0, qv1, ub0, ub1, ib0, ib1 = _scan_kernel(
        user_id, item_id,
        p.T, q.T, user_bias.T, item_bias.T)
    out = _merge_kernel(avg_score.T, pv0, pv1, qv0, qv1,
                        ub0, ub1, ib0, ib1)
    return out.reshape(B, 1)


# speedup vs baseline: 28.5559x; 1.0174x over previous
import functools

import jax
import jax.numpy as jnp
from jax import lax
from jax.experimental import pallas as pl
from jax.experimental.pallas import tpu as pltpu
from jax.experimental.pallas import tpu_sc as plsc

B = 16384
D = 32
NROWS = 1000000

_info = plsc.get_sparse_core_info()
NC = _info.num_cores
NS = _info.num_subcores
L = _info.num_lanes
NW = NC * NS
BPW = B // NW

RNG = 31232
WL = 31872
X = 18432
CAP = 704
BUF = 832
NCH = B // L
DH = D // 2

_mesh = plsc.VectorSubcoreMesh(core_axis_name="c", subcore_axis_name="s")


@functools.partial(
    pl.kernel,
    mesh=_mesh,
    compiler_params=pltpu.CompilerParams(needs_layout_passes=False),
    out_type=(
        jax.ShapeDtypeStruct((D * X,), jnp.float32),
        jax.ShapeDtypeStruct((D * X,), jnp.float32),
        jax.ShapeDtypeStruct((D * X,), jnp.float32),
        jax.ShapeDtypeStruct((D * X,), jnp.float32),
        jax.ShapeDtypeStruct((X,), jnp.float32),
        jax.ShapeDtypeStruct((X,), jnp.float32),
        jax.ShapeDtypeStruct((X,), jnp.float32),
        jax.ShapeDtypeStruct((X,), jnp.float32),
    ),
    scratch_types=[
        pltpu.VMEM((WL,), jnp.float32),
        pltpu.VMEM((WL,), jnp.float32),
        pltpu.VMEM((B,), jnp.int32),
        pltpu.VMEM((B,), jnp.int32),
        pltpu.VMEM((BUF,), jnp.int32),
        pltpu.VMEM((BUF,), jnp.int32),
        pltpu.VMEM((BUF,), jnp.int32),
        pltpu.VMEM((BUF,), jnp.int32),
        pltpu.VMEM((CAP,), jnp.int32),
        pltpu.VMEM((CAP,), jnp.float32),
        pltpu.VMEM((CAP,), jnp.int32),
        pltpu.VMEM((CAP,), jnp.float32),
        pltpu.VMEM((2304,), jnp.float32),
        pltpu.VMEM_SHARED((DH * X,), jnp.float32),
        pltpu.VMEM_SHARED((X,), jnp.float32),
        pltpu.SemaphoreType.DMA,
        pltpu.SemaphoreType.DMA,
        pltpu.SemaphoreType.DMA,
        pltpu.SemaphoreType.DMA,
    ],
)
def _scan_kernel(uid_hbm, iid_hbm, p_hbm, q_hbm, ub_hbm, ib_hbm,
                 pv0, pv1, qv0, qv1, ubp0, ubp1, ibp0, ibp1,
                 win0, win1, uidl, iidl, offs_u, bs_u, offs_i, bs_i,
                 posa, valsa, posb, valsb, zbuf, part, partb,
                 sem0, sem1, sca, scb):
    cid = lax.axis_index("c")
    sid = lax.axis_index("s")
    wid = sid * NC + cid
    lo = pl.multiple_of(wid * RNG, 128)
    hi = jnp.where(wid == NW - 1, NROWS, lo + RNG)
    lane = lax.iota(jnp.int32, L)
    shp = (DH * X) // NS
    bshp = X // NS

    def zero(g, _):
        zbuf[pl.ds(g * L, L)] = jnp.zeros((L,), jnp.float32)
        return 0
    lax.fori_loop(0, 2304 // L, zero, 0)

    pltpu.sync_copy(uid_hbm, uidl)
    pltpu.sync_copy(iid_hbm, iidl)
    pltpu.async_copy(p_hbm.at[0, pl.ds(lo, WL)], win0, sem0)
    pltpu.async_copy(p_hbm.at[1, pl.ds(lo, WL)], win1, sem1)

    def prefill(j, _):
        o = j * L
        dump = B + lax.rem(wid * 61 + o + lane, 2048)
        bs_u[pl.ds(o, L)] = dump
        bs_i[pl.ds(o, L)] = dump
        offs_u[pl.ds(o, L)] = jnp.zeros((L,), jnp.int32)
        offs_i[pl.ds(o, L)] = jnp.zeros((L,), jnp.int32)
        return 0
    lax.fori_loop(0, BUF // L, prefill, 0)

    def scan(g, carry):
        cu, ci = carry
        o = g * L
        u = uidl[pl.ds(o, L)]
        i = iidl[pl.ds(o, L)]
        b = o + lane
        mu = (u >= lo) & (u < hi)
        mi = (i >= lo) & (i < hi)
        plsc.store_compressed(offs_u.at[pl.ds(cu, L)], u - lo, mask=mu)
        plsc.store_compressed(bs_u.at[pl.ds(cu, L)], b, mask=mu)
        plsc.store_compressed(offs_i.at[pl.ds(ci, L)], i - lo, mask=mi)
        plsc.store_compressed(bs_i.at[pl.ds(ci, L)], b, mask=mi)
        cu = cu + plsc.all_reduce_population_count(mu)[0]
        ci = ci + plsc.all_reduce_population_count(mi)[0]
        return cu, ci
    lax.fori_loop(0, NCH, scan, (jnp.int32(0), jnp.int32(0)))

    def extract(win, offs, bs, pos, vals, dbase):
        for j in range(CAP // L):
            o = j * L
            idx = offs[pl.ds(o, L)]
            vals[pl.ds(o, L)] = plsc.load_gather(win, [idx])
            pos[pl.ds(o, L)] = bs[pl.ds(o, L)] + dbase

    def phase(tbl, bias, offs, bs, tout0, tout1, bout0, bout1, d0, do_bias,
              prefired=False):
        for c in range(shp // 2304):
            pltpu.sync_copy(zbuf, part.at[pl.ds(sid * shp + c * 2304, 2304)])
        if do_bias:
            pltpu.sync_copy(zbuf.at[pl.ds(0, bshp)],
                            partb.at[pl.ds(sid * bshp, bshp)])
        plsc.subcore_barrier()

        if not prefired:
            pltpu.async_copy(tbl.at[d0, pl.ds(lo, WL)], win0, sem0)
            pltpu.async_copy(tbl.at[d0 + 1, pl.ds(lo, WL)], win1, sem1)
        pltpu.make_async_copy(tbl.at[0, pl.ds(lo, WL)], win0, sem0).wait()
        extract(win0, offs, bs, posa, valsa, 0)
        pltpu.async_copy(valsa, part.at[posa], sca)
        pltpu.async_copy(tbl.at[d0 + 2, pl.ds(lo, WL)], win0, sem0)
        pltpu.make_async_copy(tbl.at[0, pl.ds(lo, WL)], win1, sem1).wait()
        extract(win1, offs, bs, posb, valsb, X)
        pltpu.async_copy(valsb, part.at[posb], scb)
        pltpu.async_copy(tbl.at[d0 + 3, pl.ds(lo, WL)], win1, sem1)

        def slab(dd, _):
            @pl.when(lax.rem(dd, 2) == 0)
            def _():
                pltpu.make_async_copy(valsa, part.at[posa], sca).wait()
                pltpu.make_async_copy(
                    tbl.at[0, pl.ds(lo, WL)], win0, sem0).wait()
                extract(win0, offs, bs, posa, valsa, dd * X)
                pltpu.async_copy(valsa, part.at[posa], sca)

                @pl.when(dd < DH - 2)
                def _():
                    pltpu.async_copy(
                        tbl.at[d0 + dd + 2, pl.ds(lo, WL)], win0, sem0)

            @pl.when(lax.rem(dd, 2) == 1)
            def _():
                pltpu.make_async_copy(valsb, part.at[posb], scb).wait()
                pltpu.make_async_copy(
                    tbl.at[0, pl.ds(lo, WL)], win1, sem1).wait()
                extract(win1, offs, bs, posb, valsb, dd * X)
                pltpu.async_copy(valsb, part.at[posb], scb)

                @pl.when(dd < DH - 2)
                def _():
                    pltpu.async_copy(
                        tbl.at[d0 + dd + 2, pl.ds(lo, WL)], win1, sem1)
            return 0
        lax.fori_loop(2, DH, slab, 0)

        pltpu.make_async_copy(valsa, part.at[posa], sca).wait()
        if do_bias:
            cbias = pltpu.async_copy(bias.at[0, pl.ds(lo, WL)], win0, sem0)
            cbias.wait()
            extract(win0, offs, bs, posa, valsa, 0)
            ca = pltpu.async_copy(valsa, partb.at[posa], sca)
        pltpu.make_async_copy(valsb, part.at[posb], scb).wait()
        if do_bias:
            ca.wait()
        plsc.subcore_barrier()

        @pl.when(cid == 0)
        def _():
            pltpu.sync_copy(part.at[pl.ds(sid * shp, shp)],
                            tout0.at[pl.ds(d0 * X + sid * shp, shp)])
            if do_bias:
                pltpu.sync_copy(partb.at[pl.ds(sid * bshp, bshp)],
                                bout0.at[pl.ds(sid * bshp, bshp)])

        @pl.when(cid == 1)
        def _():
            pltpu.sync_copy(part.at[pl.ds(sid * shp, shp)],
                            tout1.at[pl.ds(d0 * X + sid * shp, shp)])
            if do_bias:
                pltpu.sync_copy(partb.at[pl.ds(sid * bshp, bshp)],
                                bout1.at[pl.ds(sid * bshp, bshp)])
        plsc.subcore_barrier()

    phase(p_hbm, ub_hbm, offs_u, bs_u, pv0, pv1, ubp0, ubp1, 0, True,
          prefired=True)
    phase(p_hbm, ub_hbm, offs_u, bs_u, pv0, pv1, ubp0, ubp1, DH, False)
    phase(q_hbm, ib_hbm, offs_i, bs_i, qv0, qv1, ibp0, ibp1, 0, True)
    phase(q_hbm, ib_hbm, offs_i, bs_i, qv0, qv1, ibp0, ibp1, DH, False)


@functools.partial(
    pl.kernel,
    mesh=_mesh,
    compiler_params=pltpu.CompilerParams(needs_layout_passes=False),
    out_type=jax.ShapeDtypeStruct((B,), jnp.float32),
    scratch_types=[
        pltpu.VMEM((D, BPW), jnp.float32),
        pltpu.VMEM((D, BPW), jnp.float32),
        pltpu.VMEM((D, BPW), jnp.float32),
        pltpu.VMEM((D, BPW), jnp.float32),
        pltpu.VMEM((BPW,), jnp.float32),
        pltpu.VMEM((BPW,), jnp.float32),
        pltpu.VMEM((BPW,), jnp.float32),
        pltpu.VMEM((BPW,), jnp.float32),
        pltpu.VMEM((BPW,), jnp.float32),
        pltpu.VMEM((BPW,), jnp.float32),
        pltpu.SemaphoreType.DMA,
    ],
)
def _merge_kernel(avg_hbm, pv0, pv1, qv0, qv1, ubp0, ubp1, ibp0, ibp1,
                  out_hbm, p0s, p1s, q0s, q1s, u0s, u1s, i0s, i1s,
                  avgv, outv, sem):
    wid = lax.axis_index("s") * NC + lax.axis_index("c")
    base = wid * BPW

    copies = []
    for d in range(D):
        copies.append(pltpu.async_copy(
            pv0.at[pl.ds(d * X + base, BPW)], p0s.at[d], sem))
        copies.append(pltpu.async_copy(
            pv1.at[pl.ds(d * X + base, BPW)], p1s.at[d], sem))
        copies.append(pltpu.async_copy(
            qv0.at[pl.ds(d * X + base, BPW)], q0s.at[d], sem))
        copies.append(pltpu.async_copy(
            qv1.at[pl.ds(d * X + base, BPW)], q1s.at[d], sem))
    copies.append(pltpu.async_copy(ubp0.at[pl.ds(base, BPW)], u0s, sem))
    copies.append(pltpu.async_copy(ubp1.at[pl.ds(base, BPW)], u1s, sem))
    copies.append(pltpu.async_copy(ibp0.at[pl.ds(base, BPW)], i0s, sem))
    copies.append(pltpu.async_copy(ibp1.at[pl.ds(base, BPW)], i1s, sem))
    pltpu.sync_copy(avg_hbm.at[0, pl.ds(base, BPW)], avgv)
    for c in copies:
        c.wait()

    def body(g, _):
        o = g * L
        acc = (avgv[pl.ds(o, L)]
               + u0s[pl.ds(o, L)] + u1s[pl.ds(o, L)]
               + i0s[pl.ds(o, L)] + i1s[pl.ds(o, L)])
        for d in range(D):
            acc += ((p0s[d, pl.ds(o, L)] + p1s[d, pl.ds(o, L)])
                    * (q0s[d, pl.ds(o, L)] + q1s[d, pl.ds(o, L)]))
        outv[pl.ds(o, L)] = acc
        return 0

    lax.fori_loop(0, BPW // L, body, 0)
    pltpu.sync_copy(outv, out_hbm.at[pl.ds(base, BPW)])


def kernel(user_id, item_id, avg_score, p, q, user_bias, item_bias):
    pv0, pv1, qv0, qv1, ub0, ub1, ib0, ib1 = _scan_kernel(
        user_id, item_id,
        p.T, q.T, user_bias.T, item_bias.T)
    out = _merge_kernel(avg_score.T, pv0, pv1, qv0, qv1,
                        ub0, ub1, ib0, ib1)
    return out.reshape(B, 1)
